# baseline (speedup 1.0000x reference)
import functools

import jax
import jax.numpy as jnp
from jax import lax
from jax.experimental import pallas as pl
from jax.experimental.pallas import tpu as pltpu
from jax.experimental.pallas import tpu_sc as plsc


def _make_gather_mean(V, D, B, C, NW):
    b_per = B // NW
    n_idx = b_per * C
    CH = 128
    n_ch = n_idx // CH
    n_lane = D // 16

    mesh = plsc.VectorSubcoreMesh(core_axis_name="c", subcore_axis_name="s")

    @functools.partial(
        pl.kernel, mesh=mesh,
        out_type=jax.ShapeDtypeStruct((B, D), jnp.float32),
        scratch_types=[
            pltpu.VMEM((n_ch, CH), jnp.int32),
            pltpu.VMEM((n_idx, D), jnp.float32),
            pltpu.VMEM((b_per, D), jnp.float32),
            pltpu.SemaphoreType.DMA,
        ],
        compiler_params=pltpu.CompilerParams(use_tc_tiling_on_sc=False),
    )
    def gather_mean(ctx_hbm, table_hbm, out_hbm, idx_v, rows_v, acc_v, sem):
        wid = lax.axis_index("s") * 2 + lax.axis_index("c")
        pltpu.sync_copy(ctx_hbm.at[wid], idx_v)
        copies = [
            pltpu.async_copy(table_hbm.at[idx_v.at[j]],
                             rows_v.at[pl.ds(j * CH, CH)], sem)
            for j in range(n_ch)
        ]
        for cp in copies:
            cp.wait()

        scale = jnp.float32(1.0 / C)

        def body(bi, carry):
            base = bi * C
            accs = [rows_v[base, pl.ds(ch * 16, 16)] for ch in range(n_lane)]
            for j in range(1, C):
                for ch in range(n_lane):
                    accs[ch] = accs[ch] + rows_v[base + j, pl.ds(ch * 16, 16)]
            for ch in range(n_lane):
                acc_v[bi, pl.ds(ch * 16, 16)] = accs[ch] * scale
            return carry

        lax.fori_loop(0, b_per, body, 0)
        pltpu.sync_copy(acc_v, out_hbm.at[pl.ds(wid * b_per, b_per)])

    return gather_mean


def _proj_body(x_ref, w_ref, b_ref, o_ref):
    o_ref[...] = lax.dot_general(
        x_ref[...], w_ref[...], (((1,), (1,)), ((), ())),
        preferred_element_type=jnp.float32,
    ) + b_ref[...]


def _projection(x, W, b2, VB):
    B, D = x.shape
    V = W.shape[0]
    grid = (pl.cdiv(V, VB),)
    return pl.pallas_call(
        _proj_body,
        grid=grid,
        in_specs=[
            pl.BlockSpec((B, D), lambda i: (0, 0)),
            pl.BlockSpec((VB, D), lambda i: (i, 0)),
            pl.BlockSpec((1, VB), lambda i: (0, i)),
        ],
        out_specs=pl.BlockSpec((B, VB), lambda i: (0, i)),
        out_shape=jax.ShapeDtypeStruct((B, V), jnp.float32),
        compiler_params=pltpu.CompilerParams(
            dimension_semantics=("arbitrary",),
        ),
    )(x, W, b2)


def kernel(context, embeddings, W, b):
    B, C = context.shape
    V, D = embeddings.shape
    NW = 32
    ctx3 = context.astype(jnp.int32).reshape(NW, -1, 128)
    x = _make_gather_mean(V, D, B, C, NW)(ctx3, embeddings)
    return _projection(x, W, b.reshape(1, V), VB=512)


# speedup vs baseline: 1.1352x; 1.1352x over previous
import functools

import jax
import jax.numpy as jnp
from jax import lax
from jax.experimental import pallas as pl
from jax.experimental.pallas import tpu as pltpu
from jax.experimental.pallas import tpu_sc as plsc


def _make_gather_mean(V, D, B, C, NW):
    b_per = B // NW
    n_idx = b_per * C
    CH = 128
    n_ch = n_idx // CH
    n_lane = D // 16

    mesh = plsc.VectorSubcoreMesh(core_axis_name="c", subcore_axis_name="s")

    @functools.partial(
        pl.kernel, mesh=mesh,
        out_type=jax.ShapeDtypeStruct((B, D), jnp.float32),
        scratch_types=[
            pltpu.VMEM((n_ch, CH), jnp.int32),
            pltpu.VMEM((n_idx, D), jnp.float32),
            pltpu.VMEM((b_per, D), jnp.float32),
            pltpu.SemaphoreType.DMA,
        ],
        compiler_params=pltpu.CompilerParams(use_tc_tiling_on_sc=False),
    )
    def gather_mean(ctx_hbm, table_hbm, out_hbm, idx_v, rows_v, acc_v, sem):
        wid = lax.axis_index("s") * 2 + lax.axis_index("c")
        pltpu.sync_copy(ctx_hbm.at[wid], idx_v)
        copies = [
            pltpu.async_copy(table_hbm.at[idx_v.at[j]],
                             rows_v.at[pl.ds(j * CH, CH)], sem)
            for j in range(n_ch)
        ]
        for cp in copies:
            cp.wait()

        scale = jnp.float32(1.0 / C)

        def body(bi, carry):
            base = bi * C
            accs = [rows_v[base, pl.ds(ch * 16, 16)] for ch in range(n_lane)]
            for j in range(1, C):
                for ch in range(n_lane):
                    accs[ch] = accs[ch] + rows_v[base + j, pl.ds(ch * 16, 16)]
            for ch in range(n_lane):
                acc_v[bi, pl.ds(ch * 16, 16)] = accs[ch] * scale
            return carry

        lax.fori_loop(0, b_per, body, 0)
        pltpu.sync_copy(acc_v, out_hbm.at[pl.ds(wid * b_per, b_per)])

    return gather_mean


def _proj_body(x_ref, w_ref, b_ref, o_ref):
    o_ref[...] = lax.dot_general(
        x_ref[...], w_ref[...], (((1,), (1,)), ((), ())),
        preferred_element_type=jnp.float32,
    ) + b_ref[...]


def _projection(x, W, b2, VB):
    B, D = x.shape
    V = W.shape[0]
    grid = (pl.cdiv(V, VB),)
    return pl.pallas_call(
        _proj_body,
        grid=grid,
        in_specs=[
            pl.BlockSpec((B, D), lambda i: (0, 0)),
            pl.BlockSpec((VB, D), lambda i: (i, 0)),
            pl.BlockSpec((1, VB), lambda i: (0, i)),
        ],
        out_specs=pl.BlockSpec((B, VB), lambda i: (0, i)),
        out_shape=jax.ShapeDtypeStruct((B, V), jnp.float32),
        compiler_params=pltpu.CompilerParams(
            dimension_semantics=("arbitrary",),
        ),
    )(x, W, b2)


def kernel(context, embeddings, W, b):
    B, C = context.shape
    V, D = embeddings.shape
    NW = 32
    ctx3 = context.astype(jnp.int32).reshape(NW, -1, 128)
    x = _make_gather_mean(V, D, B, C, NW)(ctx3, embeddings)
    return _projection(x, W, b.reshape(1, V), VB=2048)


# speedup vs baseline: 3.1494x; 2.7744x over previous
import functools

import jax
import jax.numpy as jnp
from jax import lax
from jax.experimental import pallas as pl
from jax.experimental.pallas import tpu as pltpu
from jax.experimental.pallas import tpu_sc as plsc


def _make_gather_mean(V, D, B, C, NW):
    b_per = B // NW
    n_idx = b_per * C
    CH = 128
    n_ch = n_idx // CH
    n_lane = D // 16

    mesh = plsc.VectorSubcoreMesh(core_axis_name="c", subcore_axis_name="s")

    @functools.partial(
        pl.kernel, mesh=mesh,
        out_type=jax.ShapeDtypeStruct((B, D), jnp.float32),
        scratch_types=[
            pltpu.VMEM((n_ch, CH), jnp.int32),
            pltpu.VMEM((n_idx, D), jnp.float32),
            pltpu.VMEM((b_per, D), jnp.float32),
            pltpu.SemaphoreType.DMA,
        ],
        compiler_params=pltpu.CompilerParams(use_tc_tiling_on_sc=False),
    )
    def gather_mean(ctx_hbm, table_hbm, out_hbm, idx_v, rows_v, acc_v, sem):
        wid = lax.axis_index("s") * 2 + lax.axis_index("c")
        pltpu.sync_copy(ctx_hbm.at[wid], idx_v)
        copies = [
            pltpu.async_copy(table_hbm.at[idx_v.at[j]],
                             rows_v.at[pl.ds(j * CH, CH)], sem)
            for j in range(n_ch)
        ]
        for cp in copies:
            cp.wait()

        scale = jnp.float32(1.0 / C)

        def body(bi, carry):
            base = bi * C
            accs = [rows_v[base, pl.ds(ch * 16, 16)] for ch in range(n_lane)]
            for j in range(1, C):
                for ch in range(n_lane):
                    accs[ch] = accs[ch] + rows_v[base + j, pl.ds(ch * 16, 16)]
            for ch in range(n_lane):
                acc_v[bi, pl.ds(ch * 16, 16)] = accs[ch] * scale
            return carry

        lax.fori_loop(0, b_per, body, 0)
        pltpu.sync_copy(acc_v, out_hbm.at[pl.ds(wid * b_per, b_per)])

    return gather_mean


def _proj_body(x_ref, wt_ref, b_ref, o_ref):
    vb = o_ref.shape[0]
    o_ref[...] = lax.dot_general(
        wt_ref[...], x_ref[...], (((0,), (1,)), ((), ())),
        preferred_element_type=jnp.float32,
    ) + jnp.reshape(b_ref[...], (vb, 1))


def _projection_t(x, WT, b2, VB):
    B, D = x.shape
    V = WT.shape[1]
    grid = (pl.cdiv(V, VB),)
    return pl.pallas_call(
        _proj_body,
        grid=grid,
        in_specs=[
            pl.BlockSpec((B, D), lambda i: (0, 0)),
            pl.BlockSpec((D, VB), lambda i: (0, i)),
            pl.BlockSpec((VB,), lambda i: (i,)),
        ],
        out_specs=pl.BlockSpec((VB, B), lambda i: (i, 0)),
        out_shape=jax.ShapeDtypeStruct((V, B), jnp.float32),
        compiler_params=pltpu.CompilerParams(
            dimension_semantics=("arbitrary",),
        ),
    )(x, WT, b2)


def kernel(context, embeddings, W, b):
    B, C = context.shape
    V, D = embeddings.shape
    NW = 32
    ctx3 = context.astype(jnp.int32).reshape(NW, -1, 128)
    x = _make_gather_mean(V, D, B, C, NW)(ctx3, embeddings)
    outT = _projection_t(x, W.T, b, VB=2048)
    return outT.T


# speedup vs baseline: 3.2687x; 1.0379x over previous
import functools

import jax
import jax.numpy as jnp
from jax import lax
from jax.experimental import pallas as pl
from jax.experimental.pallas import tpu as pltpu
from jax.experimental.pallas import tpu_sc as plsc


def _relayout_body(embt_ref, o_ref):
    d = embt_ref.shape[0]
    eye = (lax.broadcasted_iota(jnp.int32, (d, d), 0)
           == lax.broadcasted_iota(jnp.int32, (d, d), 1)).astype(jnp.float32)
    t = lax.dot_general(embt_ref[...], eye, (((0,), (0,)), ((), ())),
                        preferred_element_type=jnp.float32)
    o_ref[...] = jnp.concatenate([t, t], axis=1)


def _relayout(embT, VT):
    D, V = embT.shape
    return pl.pallas_call(
        _relayout_body,
        grid=(pl.cdiv(V, VT),),
        in_specs=[pl.BlockSpec((D, VT), lambda i: (0, i))],
        out_specs=pl.BlockSpec((VT, 2 * D), lambda i: (i, 0)),
        out_shape=jax.ShapeDtypeStruct((V, 2 * D), jnp.float32),
        compiler_params=pltpu.CompilerParams(
            dimension_semantics=("arbitrary",),
        ),
    )(embT)


def _make_gather_mean(V, B, C, NW):
    b_per = B // NW
    n_idx = b_per * C
    CH = 128
    n_ch = n_idx // CH

    mesh = plsc.VectorSubcoreMesh(core_axis_name="c", subcore_axis_name="s")

    @functools.partial(
        pl.kernel, mesh=mesh,
        out_type=jax.ShapeDtypeStruct((B, 64), jnp.float32),
        scratch_types=[
            pltpu.VMEM((n_ch, CH), jnp.int32),
            pltpu.VMEM((n_idx, 128), jnp.float32),
            pltpu.VMEM((b_per, 64), jnp.float32),
            pltpu.SemaphoreType.DMA,
        ],
        compiler_params=pltpu.CompilerParams(use_tc_tiling_on_sc=True),
    )
    def gather_mean(ctx_hbm, table_hbm, out_hbm, idx_v, rows_v, acc_v, sem):
        wid = lax.axis_index("s") * 2 + lax.axis_index("c")
        pltpu.sync_copy(ctx_hbm.at[wid], idx_v)
        copies = [
            pltpu.async_copy(table_hbm.at[idx_v.at[j]],
                             rows_v.at[pl.ds(j * CH, CH)], sem)
            for j in range(n_ch)
        ]
        for cp in copies:
            cp.wait()

        scale = jnp.float32(1.0 / C)

        def body(bi, carry):
            base = bi * C
            accs = [rows_v[base, pl.ds(ch * 16, 16)] for ch in range(4)]
            for j in range(1, C):
                for ch in range(4):
                    accs[ch] = accs[ch] + rows_v[base + j, pl.ds(ch * 16, 16)]
            for ch in range(4):
                acc_v[bi, pl.ds(ch * 16, 16)] = accs[ch] * scale
            return carry

        lax.fori_loop(0, b_per, body, 0)
        pltpu.sync_copy(acc_v, out_hbm.at[pl.ds(wid * b_per, b_per)])

    return gather_mean


def _proj_body(x_ref, wt_ref, b_ref, o_ref):
    vb = o_ref.shape[0]
    o_ref[...] = lax.dot_general(
        wt_ref[...], x_ref[...], (((0,), (1,)), ((), ())),
        preferred_element_type=jnp.float32,
    ) + jnp.reshape(b_ref[...], (vb, 1))


def _projection_t(x, WT, b, VB):
    B, D = x.shape
    V = WT.shape[1]
    grid = (pl.cdiv(V, VB),)
    return pl.pallas_call(
        _proj_body,
        grid=grid,
        in_specs=[
            pl.BlockSpec((B, D), lambda i: (0, 0)),
            pl.BlockSpec((D, VB), lambda i: (0, i)),
            pl.BlockSpec((VB,), lambda i: (i,)),
        ],
        out_specs=pl.BlockSpec((VB, B), lambda i: (i, 0)),
        out_shape=jax.ShapeDtypeStruct((V, B), jnp.float32),
        compiler_params=pltpu.CompilerParams(
            dimension_semantics=("arbitrary",),
        ),
    )(x, WT, b)


def kernel(context, embeddings, W, b):
    B, C = context.shape
    V, D = embeddings.shape
    NW = 32
    ctx3 = context.astype(jnp.int32).reshape(NW, -1, 128)
    emb128 = _relayout(embeddings.T, VT=2048)
    x = _make_gather_mean(V, B, C, NW)(ctx3, emb128)
    outT = _projection_t(x, W.T, b, VB=2048)
    return outT.T


# speedup vs baseline: 3.4635x; 1.0596x over previous
import functools

import jax
import jax.numpy as jnp
from jax import lax
from jax.experimental import pallas as pl
from jax.experimental.pallas import tpu as pltpu
from jax.experimental.pallas import tpu_sc as plsc


def _relayout_body(top_ref, bot_ref, o_ref):
    d = top_ref.shape[0]
    eye = (lax.broadcasted_iota(jnp.int32, (d, d), 0)
           == lax.broadcasted_iota(jnp.int32, (d, d), 1)).astype(jnp.float32)
    dn = (((0,), (0,)), ((), ()))
    t = lax.dot_general(top_ref[...], eye, dn, preferred_element_type=jnp.float32)
    u = lax.dot_general(bot_ref[...], eye, dn, preferred_element_type=jnp.float32)
    o_ref[...] = jnp.concatenate([t, u], axis=1)


def _relayout(embT, VT, K):
    D, V = embT.shape
    n = K // VT
    last = pl.cdiv(V, VT) - 1
    return pl.pallas_call(
        _relayout_body,
        grid=(n,),
        in_specs=[
            pl.BlockSpec((D, VT), lambda i: (0, i)),
            pl.BlockSpec((D, VT), lambda i, n=n, last=last:
                         (0, jnp.minimum(i + n, last))),
        ],
        out_specs=pl.BlockSpec((VT, 2 * D), lambda i: (i, 0)),
        out_shape=jax.ShapeDtypeStruct((K, 2 * D), jnp.float32),
        compiler_params=pltpu.CompilerParams(
            dimension_semantics=("arbitrary",),
        ),
    )(embT, embT)


def _make_gather_mean(B, C, NW):
    b_per = B // NW
    n_idx = b_per * C
    CH = 128
    n_ch = n_idx // CH
    n_par_rows = (n_idx * 16) // 128

    mesh = plsc.VectorSubcoreMesh(core_axis_name="c", subcore_axis_name="s")

    @functools.partial(
        pl.kernel, mesh=mesh,
        out_type=jax.ShapeDtypeStruct((B, 64), jnp.float32),
        scratch_types=[
            pltpu.VMEM((n_ch, CH), jnp.int32),
            pltpu.VMEM((n_par_rows, 128), jnp.float32),
            pltpu.VMEM((n_idx, 128), jnp.float32),
            pltpu.VMEM((b_per, 64), jnp.float32),
            pltpu.SemaphoreType.DMA,
        ],
        compiler_params=pltpu.CompilerParams(use_tc_tiling_on_sc=True),
    )
    def gather_mean(ctx_hbm, par_hbm, table_hbm, out_hbm,
                    idx_v, par_v, rows_v, acc_v, sem):
        wid = lax.axis_index("s") * 2 + lax.axis_index("c")
        pltpu.sync_copy(ctx_hbm.at[wid], idx_v)
        pltpu.sync_copy(par_hbm.at[wid], par_v)
        copies = [
            pltpu.async_copy(table_hbm.at[idx_v.at[j]],
                             rows_v.at[pl.ds(j * CH, CH)], sem)
            for j in range(n_ch)
        ]
        for cp in copies:
            cp.wait()

        scale = jnp.float32(1.0 / C)

        def body(bi, carry):
            base = bi * C
            accs = [jnp.zeros((16,), jnp.float32) for _ in range(4)]
            for j in range(C):
                flat = (base + j) * 16
                w = par_v[flat >> 7, pl.ds(flat & 127, 16)]
                for ch in range(4):
                    lo = rows_v[base + j, pl.ds(ch * 16, 16)]
                    hi = rows_v[base + j, pl.ds(64 + ch * 16, 16)]
                    accs[ch] = accs[ch] + (lo + (hi - lo) * w)
            for ch in range(4):
                acc_v[bi, pl.ds(ch * 16, 16)] = accs[ch] * scale
            return carry

        lax.fori_loop(0, b_per, body, 0)
        pltpu.sync_copy(acc_v, out_hbm.at[pl.ds(wid * b_per, b_per)])

    return gather_mean


def _proj_body(x_ref, wt_ref, b_ref, o_ref):
    vb = o_ref.shape[0]
    o_ref[...] = lax.dot_general(
        wt_ref[...], x_ref[...], (((0,), (1,)), ((), ())),
        preferred_element_type=jnp.float32,
    ) + jnp.reshape(b_ref[...], (vb, 1))


def _projection_t(x, WT, b, VB):
    B, D = x.shape
    V = WT.shape[1]
    grid = (pl.cdiv(V, VB),)
    return pl.pallas_call(
        _proj_body,
        grid=grid,
        in_specs=[
            pl.BlockSpec((B, D), lambda i: (0, 0)),
            pl.BlockSpec((D, VB), lambda i: (0, i)),
            pl.BlockSpec((VB,), lambda i: (i,)),
        ],
        out_specs=pl.BlockSpec((VB, B), lambda i: (i, 0)),
        out_shape=jax.ShapeDtypeStruct((V, B), jnp.float32),
        compiler_params=pltpu.CompilerParams(
            dimension_semantics=("arbitrary",),
        ),
    )(x, WT, b)


def kernel(context, embeddings, W, b):
    B, C = context.shape
    V, D = embeddings.shape
    NW = 32
    K = 51200
    ctx = context.astype(jnp.int32)
    idx3 = jnp.where(ctx < K, ctx, ctx - K).reshape(NW, -1, 128)
    par = (ctx >= K).astype(jnp.float32).reshape(NW, -1, 1)
    par16 = jnp.broadcast_to(par, par.shape[:2] + (16,)).reshape(NW, -1, 128)
    pairs = _relayout(embeddings.T, VT=2048, K=K)
    x = _make_gather_mean(B, C, NW)(idx3, par16, pairs)
    outT = _projection_t(x, W.T, b, VB=2048)
    return outT.T
